# baseline (speedup 1.0000x reference)
import functools
import numpy as np
import jax
import jax.numpy as jnp
from jax import lax
from jax.experimental import pallas as pl
from jax.experimental.pallas import tpu as pltpu
from jax.experimental.pallas import tpu_sc as plsc

D_MODEL = 256
N_LEVELS = 4
N_HEADS = 8
N_POINTS = 4
D_HEAD = D_MODEL // N_HEADS
SPATIAL_NP = np.array([[64, 64], [32, 32], [16, 16], [8, 8]], dtype=np.int32)
LEN_IN = int((SPATIAL_NP[:, 0] * SPATIAL_NP[:, 1]).sum())
B_SZ = 2
LEN_Q = LEN_IN
N_LANE = N_HEADS * N_LEVELS * N_POINTS
STARTS_NP = np.concatenate([[0], np.cumsum(SPATIAL_NP[:, 0] * SPATIAL_NP[:, 1])[:-1]])

_lane = np.arange(N_LANE)
_h_lane = (_lane // (N_LEVELS * N_POINTS)).astype(np.int32)
_l_lane = (_lane // N_POINTS) % N_LEVELS
W_LANE = SPATIAL_NP[_l_lane, 1].astype(np.float32).reshape(1, N_LANE)
H_LANE = SPATIAL_NP[_l_lane, 0].astype(np.float32).reshape(1, N_LANE)
WI_LANE = SPATIAL_NP[_l_lane, 1].astype(np.int32).reshape(1, N_LANE)
HI_LANE = SPATIAL_NP[_l_lane, 0].astype(np.int32).reshape(1, N_LANE)
BASE_LANE = STARTS_NP[_l_lane].astype(np.int32).reshape(1, N_LANE)
H_LANE_I = _h_lane.reshape(1, N_LANE)
BD_NP = np.kron(np.eye(N_HEADS, dtype=np.float32),
                np.ones((N_LEVELS * N_POINTS, N_LEVELS * N_POINTS), np.float32))
_k = np.arange(D_MODEL)
_r = _k % D_HEAD
VAL_PERM = (_k // D_HEAD) * D_HEAD + np.where(_r % 2 == 0, _r // 2, D_HEAD // 2 + _r // 2)

QB = 544
NB = (B_SZ * LEN_Q) // QB

SC_CORES = 2
SC_SUBCORES = 16
N_WORKERS = SC_CORES * SC_SUBCORES
ROWS_TOTAL = B_SZ * LEN_Q
ROWS_PER_W = ROWS_TOTAL // N_WORKERS
QC = 5
CHUNKS = ROWS_PER_W // QC
GROUPS = QC * 4


def _precompute_body(q_ref, if_ref, rpx_ref, rpy_ref,
                     wvx_ref, bv_ref, wox_ref, box_ref, woy_ref, boy_ref,
                     wa_ref, ba_ref, bd_ref, lanef_ref, lanei_ref,
                     val_ref, idx_ref, wgt_ref):
    blk = pl.program_id(0)
    qb = q_ref[...]
    val_ref[...] = (jnp.dot(if_ref[...], wvx_ref[...],
                            preferred_element_type=jnp.float32)
                    + bv_ref[...]).astype(jnp.bfloat16)
    offx = jnp.dot(qb, wox_ref[...], preferred_element_type=jnp.float32) + box_ref[...]
    offy = jnp.dot(qb, woy_ref[...], preferred_element_type=jnp.float32) + boy_ref[...]
    logit = jnp.dot(qb, wa_ref[...], preferred_element_type=jnp.float32) + ba_ref[...]
    e = jnp.exp(logit)
    denom = jnp.dot(e, bd_ref[...], preferred_element_type=jnp.float32)
    attnw = e / denom

    wl = lanef_ref[0:1, :]
    hl = lanef_ref[1:2, :]
    wli = lanei_ref[0:1, :]
    hli = lanei_ref[1:2, :]
    base = lanei_ref[2:3, :]
    hlane = lanei_ref[3:4, :]

    x = rpx_ref[...] * wl + offx - 0.5
    y = rpy_ref[...] * hl + offy - 0.5
    x0 = jnp.floor(x)
    y0 = jnp.floor(y)
    fx = x - x0
    fy = y - y0
    x0i = x0.astype(jnp.int32)
    y0i = y0.astype(jnp.int32)
    b = blk // (NB // B_SZ)
    row_off = b * LEN_IN * N_HEADS

    corners = ((0, 0), (0, 1), (1, 0), (1, 1))
    for c, (dy, dx) in enumerate(corners):
        xi = x0i + dx
        yi = y0i + dy
        valid = ((xi >= 0) & (xi < wli) & (yi >= 0) & (yi < hli))
        xc = jnp.clip(xi, 0, wli - 1)
        yc = jnp.clip(yi, 0, hli - 1)
        gidx = (base + yc * wli + xc) * N_HEADS + hlane + row_off
        wx = fx if dx == 1 else 1.0 - fx
        wy = fy if dy == 1 else 1.0 - fy
        w = attnw * wy * wx * valid.astype(jnp.float32)
        wu = jax.lax.bitcast_convert_type(w.astype(jnp.bfloat16),
                                          jnp.uint16).astype(jnp.uint32)
        idx_ref[:, c, :] = gidx
        wgt_ref[:, c, :] = (wu * jnp.uint32(65537)).astype(jnp.int32)


def _run_precompute(query_f, iff, rpx_f, rpy_f, W_val, b_val,
                    W_off_x, b_off_x, W_off_y, b_off_y, W_attn, b_attn, bd,
                    lanef, lanei):
    full = lambda shape: pl.BlockSpec(shape, lambda i: tuple(0 for _ in shape))
    rowblk = lambda w: pl.BlockSpec((QB, w), lambda i: (i, 0))
    return pl.pallas_call(
        _precompute_body,
        grid=(NB,),
        in_specs=[
            rowblk(D_MODEL),
            rowblk(D_MODEL),
            rowblk(N_LANE),
            rowblk(N_LANE),
            full((D_MODEL, D_MODEL)), full((1, D_MODEL)),
            full((D_MODEL, N_LANE)), full((1, N_LANE)),
            full((D_MODEL, N_LANE)), full((1, N_LANE)),
            full((D_MODEL, N_LANE)), full((1, N_LANE)),
            full((N_LANE, N_LANE)),
            full((2, N_LANE)), full((4, N_LANE)),
        ],
        out_specs=[
            rowblk(D_MODEL),
            pl.BlockSpec((QB, 4, N_LANE), lambda i: (i, 0, 0)),
            pl.BlockSpec((QB, 4, N_LANE), lambda i: (i, 0, 0)),
        ],
        out_shape=[
            jax.ShapeDtypeStruct((B_SZ * LEN_IN, D_MODEL), jnp.bfloat16),
            jax.ShapeDtypeStruct((B_SZ * LEN_Q, 4, N_LANE), jnp.int32),
            jax.ShapeDtypeStruct((B_SZ * LEN_Q, 4, N_LANE), jnp.int32),
        ],
    )(query_f, iff, rpx_f, rpy_f, W_val, b_val,
      W_off_x, b_off_x, W_off_y, b_off_y, W_attn, b_attn, bd, lanef, lanei)


def _sc_body(val_hbm, idx_hbm, wgt_hbm, out_hbm,
             idx_v, wgt_v, rows_v, out_v, sem_g, sem_iw, sem_out):
    wid = lax.axis_index("s") * SC_CORES + lax.axis_index("c")
    row0 = wid * ROWS_PER_W

    def rbase(ci):
        return row0 + jnp.minimum(ci, CHUNKS - 1) * QC

    def start_iw(ci, p):
        rr = rbase(ci)
        pltpu.async_copy(idx_hbm.at[pl.ds(rr * 4, GROUPS)], idx_v.at[p], sem_iw)
        pltpu.async_copy(wgt_hbm.at[pl.ds(rr * 4, GROUPS)], wgt_v.at[p], sem_iw)

    def wait_iw(ci, p):
        rr = rbase(ci)
        pltpu.make_async_copy(idx_hbm.at[pl.ds(rr * 4, GROUPS)], idx_v.at[p],
                              sem_iw).wait()
        pltpu.make_async_copy(wgt_hbm.at[pl.ds(rr * 4, GROUPS)], wgt_v.at[p],
                              sem_iw).wait()

    def start_gather(p):
        for g in range(GROUPS):
            pltpu.async_copy(val_hbm.at[idx_v.at[p, g]], rows_v.at[p, g], sem_g)

    def wait_gather(p):
        for g in range(GROUPS):
            pltpu.make_async_copy(val_hbm.at[idx_v.at[p, g]], rows_v.at[p, g],
                                  sem_g).wait()

    def start_out(ci, p):
        pltpu.async_copy(out_v.at[p], out_hbm.at[pl.ds(rbase(ci), QC)], sem_out)

    def wait_out(ci, p):
        pltpu.make_async_copy(out_v.at[p], out_hbm.at[pl.ds(rbase(ci), QC)],
                              sem_out).wait()

    def compute(p):
        def q_body(qq, _):
            for h in range(N_HEADS):
                acc0 = jnp.zeros((16,), jnp.float32)
                acc1 = jnp.zeros((16,), jnp.float32)
                for c in range(4):
                    g = qq * 4 + c
                    wv = wgt_v[p, g, pl.ds(h * 16, 16)]
                    acca = jnp.zeros((32,), jnp.bfloat16)
                    accb = jnp.zeros((32,), jnp.bfloat16)
                    for lp in range(N_LEVELS * N_POINTS):
                        j = h * 16 + lp
                        wb32 = plsc.bitcast(jnp.full((16,), wv[lp], jnp.int32),
                                            jnp.bfloat16)
                        prod = wb32 * rows_v[p, g, j, pl.ds(0, 32)]
                        if lp % 2 == 0:
                            acca = acca + prod
                        else:
                            accb = accb + prod
                    lo, hi = plsc.unpack(acca + accb,
                                         format=plsc.PackFormat.INTERLEAVED)
                    acc0 = acc0 + lo
                    acc1 = acc1 + hi
                out_v[p, qq, pl.ds(h * 32, 16)] = acc0
                out_v[p, qq, pl.ds(h * 32 + 16, 16)] = acc1
            return 0

        lax.fori_loop(0, QC, q_body, 0)

    start_iw(0, 0)
    wait_iw(0, 0)
    start_gather(0)
    start_iw(1, 1)

    def chunk_body(ci, _):
        p = lax.rem(ci, 2)
        pn = lax.rem(ci + 1, 2)
        wait_gather(p)
        wait_iw(ci + 1, pn)
        start_gather(pn)

        @pl.when(ci >= 2)
        def _():
            wait_out(ci - 2, p)

        compute(p)
        start_out(ci, p)
        start_iw(ci + 2, p)
        return 0

    lax.fori_loop(0, CHUNKS, chunk_body, 0)

    p_end = lax.rem(jnp.int32(CHUNKS), 2)
    wait_gather(p_end)
    wait_iw(CHUNKS, lax.rem(jnp.int32(CHUNKS + 1), 2))
    wait_out(CHUNKS - 2, lax.rem(jnp.int32(CHUNKS - 2), 2))
    wait_out(CHUNKS - 1, lax.rem(jnp.int32(CHUNKS - 1), 2))


def _run_sc(val_rows, idx, wgt):
    mesh = plsc.VectorSubcoreMesh(core_axis_name="c", subcore_axis_name="s",
                                  num_cores=SC_CORES, num_subcores=SC_SUBCORES)
    kern = pl.kernel(
        _sc_body,
        out_type=jax.ShapeDtypeStruct((B_SZ * LEN_Q, D_MODEL), jnp.float32),
        mesh=mesh,
        compiler_params=pltpu.CompilerParams(use_tc_tiling_on_sc=False,
                                             needs_layout_passes=False),
        scratch_types=[
            pltpu.VMEM((2, GROUPS, N_LANE), jnp.int32),
            pltpu.VMEM((2, GROUPS, N_LANE), jnp.int32),
            pltpu.VMEM((2, GROUPS, N_LANE, D_HEAD), jnp.bfloat16),
            pltpu.VMEM((2, QC, D_MODEL), jnp.float32),
            pltpu.SemaphoreType.DMA,
            pltpu.SemaphoreType.DMA,
            pltpu.SemaphoreType.DMA,
        ],
    )
    return kern(val_rows, idx, wgt)


def _outproj_body(x_ref, w_ref, b_ref, o_ref):
    o_ref[...] = jnp.dot(x_ref[...], w_ref[...],
                         preferred_element_type=jnp.float32) + b_ref[...]


def _run_outproj(x, W_out, b_out):
    full = lambda shape: pl.BlockSpec(shape, lambda i: tuple(0 for _ in shape))
    return pl.pallas_call(
        _outproj_body,
        grid=(NB,),
        in_specs=[pl.BlockSpec((QB, D_MODEL), lambda i: (i, 0)),
                  full((D_MODEL, D_MODEL)), full((1, D_MODEL))],
        out_specs=pl.BlockSpec((QB, D_MODEL), lambda i: (i, 0)),
        out_shape=jax.ShapeDtypeStruct((B_SZ * LEN_Q, D_MODEL), jnp.float32),
    )(x, W_out, b_out)


@jax.jit
def kernel(query, reference_points, input_flatten, input_spatial_shapes,
           input_level_start_index, W_off, b_off, W_attn, b_attn,
           W_val, b_val, W_out, b_out):
    W_off3 = W_off.reshape(D_MODEL, N_LANE, 2)
    W_off_x = W_off3[..., 0]
    W_off_y = W_off3[..., 1]
    b_off2 = b_off.reshape(N_LANE, 2)
    b_off_x = b_off2[:, 0].reshape(1, N_LANE)
    b_off_y = b_off2[:, 1].reshape(1, N_LANE)
    rp_l = jnp.broadcast_to(
        reference_points[:, :, None, :, None, :],
        (B_SZ, LEN_Q, N_HEADS, N_LEVELS, N_POINTS, 2),
    ).reshape(B_SZ * LEN_Q, N_LANE, 2)
    rpx_f = rp_l[..., 0]
    rpy_f = rp_l[..., 1]
    query_f = query.reshape(B_SZ * LEN_Q, D_MODEL)
    iff = input_flatten.reshape(B_SZ * LEN_IN, D_MODEL)
    bd = jnp.asarray(BD_NP)
    lanef = jnp.concatenate([jnp.asarray(W_LANE), jnp.asarray(H_LANE)], axis=0)
    lanei = jnp.concatenate([jnp.asarray(WI_LANE), jnp.asarray(HI_LANE),
                             jnp.asarray(BASE_LANE), jnp.asarray(H_LANE_I)], axis=0)

    perm = jnp.asarray(VAL_PERM)
    W_val_p = W_val[:, perm]
    b_val_p = b_val[perm]

    val, idx, wgt = _run_precompute(
        query_f, iff, rpx_f, rpy_f, W_val_p, b_val_p.reshape(1, D_MODEL),
        W_off_x, b_off_x, W_off_y, b_off_y,
        W_attn, b_attn.reshape(1, N_LANE), bd, lanef, lanei)

    val_rows = val.reshape(B_SZ * LEN_IN * N_HEADS, D_HEAD)
    idx_f = idx.reshape(B_SZ * LEN_Q * 4, N_LANE)
    wgt_f = wgt.reshape(B_SZ * LEN_Q * 4, N_LANE)
    sc_out = _run_sc(val_rows, idx_f, wgt_f)

    out = _run_outproj(sc_out, W_out, b_out.reshape(1, D_MODEL))
    return out.reshape(B_SZ, LEN_Q, D_MODEL)


# speedup vs baseline: 1.0420x; 1.0420x over previous
import functools
import numpy as np
import jax
import jax.numpy as jnp
from jax import lax
from jax.experimental import pallas as pl
from jax.experimental.pallas import tpu as pltpu
from jax.experimental.pallas import tpu_sc as plsc

D_MODEL = 256
N_LEVELS = 4
N_HEADS = 8
N_POINTS = 4
D_HEAD = D_MODEL // N_HEADS
SPATIAL_NP = np.array([[64, 64], [32, 32], [16, 16], [8, 8]], dtype=np.int32)
LEN_IN = int((SPATIAL_NP[:, 0] * SPATIAL_NP[:, 1]).sum())
B_SZ = 2
LEN_Q = LEN_IN
N_LANE = N_HEADS * N_LEVELS * N_POINTS
STARTS_NP = np.concatenate([[0], np.cumsum(SPATIAL_NP[:, 0] * SPATIAL_NP[:, 1])[:-1]])

_lane = np.arange(N_LANE)
_h_lane = (_lane // (N_LEVELS * N_POINTS)).astype(np.int32)
_l_lane = (_lane // N_POINTS) % N_LEVELS
W_LANE = SPATIAL_NP[_l_lane, 1].astype(np.float32).reshape(1, N_LANE)
H_LANE = SPATIAL_NP[_l_lane, 0].astype(np.float32).reshape(1, N_LANE)
WI_LANE = SPATIAL_NP[_l_lane, 1].astype(np.int32).reshape(1, N_LANE)
HI_LANE = SPATIAL_NP[_l_lane, 0].astype(np.int32).reshape(1, N_LANE)
BASE_LANE = STARTS_NP[_l_lane].astype(np.int32).reshape(1, N_LANE)
H_LANE_I = _h_lane.reshape(1, N_LANE)
BD_NP = np.kron(np.eye(N_HEADS, dtype=np.float32),
                np.ones((N_LEVELS * N_POINTS, N_LEVELS * N_POINTS), np.float32))
_k = np.arange(D_MODEL)
_r = _k % D_HEAD
VAL_PERM = (_k // D_HEAD) * D_HEAD + np.where(_r % 2 == 0, _r // 2, D_HEAD // 2 + _r // 2)

QB = 544
NB = (B_SZ * LEN_Q) // QB

SC_CORES = 2
SC_SUBCORES = 16
N_WORKERS = SC_CORES * SC_SUBCORES
ROWS_TOTAL = B_SZ * LEN_Q
ROWS_PER_W = ROWS_TOTAL // N_WORKERS
QC = 5
CHUNKS = ROWS_PER_W // QC
GROUPS = QC * 4


def _precompute_body(q_ref, if_ref, rpx_ref, rpy_ref,
                     wvx_ref, bv_ref, wox_ref, box_ref, woy_ref, boy_ref,
                     wa_ref, ba_ref, bd_ref, lanef_ref, lanei_ref,
                     val_ref, idx_ref, wgt_ref):
    blk = pl.program_id(0)
    qb = q_ref[...]
    val_ref[...] = (jnp.dot(if_ref[...], wvx_ref[...],
                            preferred_element_type=jnp.float32)
                    + bv_ref[...]).astype(jnp.bfloat16)
    offx = jnp.dot(qb, wox_ref[...], preferred_element_type=jnp.float32) + box_ref[...]
    offy = jnp.dot(qb, woy_ref[...], preferred_element_type=jnp.float32) + boy_ref[...]
    logit = jnp.dot(qb, wa_ref[...], preferred_element_type=jnp.float32) + ba_ref[...]
    e = jnp.exp(logit)
    denom = jnp.dot(e, bd_ref[...], preferred_element_type=jnp.float32)
    attnw = e / denom

    wl = lanef_ref[0:1, :]
    hl = lanef_ref[1:2, :]
    wli = lanei_ref[0:1, :]
    hli = lanei_ref[1:2, :]
    base = lanei_ref[2:3, :]
    hlane = lanei_ref[3:4, :]

    x = rpx_ref[...] * wl + offx - 0.5
    y = rpy_ref[...] * hl + offy - 0.5
    x0 = jnp.floor(x)
    y0 = jnp.floor(y)
    fx = x - x0
    fy = y - y0
    x0i = x0.astype(jnp.int32)
    y0i = y0.astype(jnp.int32)
    b = blk // (NB // B_SZ)
    row_off = b * LEN_IN * N_HEADS

    corners = ((0, 0), (0, 1), (1, 0), (1, 1))
    for c, (dy, dx) in enumerate(corners):
        xi = x0i + dx
        yi = y0i + dy
        valid = ((xi >= 0) & (xi < wli) & (yi >= 0) & (yi < hli))
        xc = jnp.clip(xi, 0, wli - 1)
        yc = jnp.clip(yi, 0, hli - 1)
        gidx = (base + yc * wli + xc) * N_HEADS + hlane + row_off
        wx = fx if dx == 1 else 1.0 - fx
        wy = fy if dy == 1 else 1.0 - fy
        w = attnw * wy * wx * valid.astype(jnp.float32)
        wu = jax.lax.bitcast_convert_type(w.astype(jnp.bfloat16),
                                          jnp.uint16).astype(jnp.uint32)
        idx_ref[c] = gidx
        wgt_ref[c] = (wu * jnp.uint32(65537)).astype(jnp.int32)


def _run_precompute(query_f, iff, rpx_f, rpy_f, W_val, b_val,
                    W_off_x, b_off_x, W_off_y, b_off_y, W_attn, b_attn, bd,
                    lanef, lanei):
    full = lambda shape: pl.BlockSpec(shape, lambda i: tuple(0 for _ in shape))
    rowblk = lambda w: pl.BlockSpec((QB, w), lambda i: (i, 0))
    return pl.pallas_call(
        _precompute_body,
        grid=(NB,),
        in_specs=[
            rowblk(D_MODEL),
            rowblk(D_MODEL),
            rowblk(N_LANE),
            rowblk(N_LANE),
            full((D_MODEL, D_MODEL)), full((1, D_MODEL)),
            full((D_MODEL, N_LANE)), full((1, N_LANE)),
            full((D_MODEL, N_LANE)), full((1, N_LANE)),
            full((D_MODEL, N_LANE)), full((1, N_LANE)),
            full((N_LANE, N_LANE)),
            full((2, N_LANE)), full((4, N_LANE)),
        ],
        out_specs=[
            rowblk(D_MODEL),
            pl.BlockSpec((4, QB, N_LANE), lambda i: (0, i, 0)),
            pl.BlockSpec((4, QB, N_LANE), lambda i: (0, i, 0)),
        ],
        out_shape=[
            jax.ShapeDtypeStruct((B_SZ * LEN_IN, D_MODEL), jnp.bfloat16),
            jax.ShapeDtypeStruct((4, B_SZ * LEN_Q, N_LANE), jnp.int32),
            jax.ShapeDtypeStruct((4, B_SZ * LEN_Q, N_LANE), jnp.int32),
        ],
    )(query_f, iff, rpx_f, rpy_f, W_val, b_val,
      W_off_x, b_off_x, W_off_y, b_off_y, W_attn, b_attn, bd, lanef, lanei)


def _sc_body(val_hbm, idx_hbm, wgt_hbm, out_hbm,
             idx_v, wgt_v, rows_v, out_v, sem_g, sem_iw, sem_out):
    wid = lax.axis_index("s") * SC_CORES + lax.axis_index("c")
    row0 = wid * ROWS_PER_W

    def rbase(ci):
        return row0 + jnp.minimum(ci, CHUNKS - 1) * QC

    def start_iw(ci, p):
        rr = rbase(ci)
        for c in range(4):
            pltpu.async_copy(idx_hbm.at[c, pl.ds(rr, QC)], idx_v.at[p, c], sem_iw)
            pltpu.async_copy(wgt_hbm.at[c, pl.ds(rr, QC)], wgt_v.at[p, c], sem_iw)

    def wait_iw(ci, p):
        rr = rbase(ci)
        for c in range(4):
            pltpu.make_async_copy(idx_hbm.at[c, pl.ds(rr, QC)], idx_v.at[p, c],
                                  sem_iw).wait()
            pltpu.make_async_copy(wgt_hbm.at[c, pl.ds(rr, QC)], wgt_v.at[p, c],
                                  sem_iw).wait()

    def start_gather(p):
        for c in range(4):
            for qq in range(QC):
                pltpu.async_copy(val_hbm.at[idx_v.at[p, c, qq]],
                                 rows_v.at[p, c * QC + qq], sem_g)

    def wait_gather(p):
        for c in range(4):
            for qq in range(QC):
                pltpu.make_async_copy(val_hbm.at[idx_v.at[p, c, qq]],
                                      rows_v.at[p, c * QC + qq], sem_g).wait()

    def start_out(ci, p):
        pltpu.async_copy(out_v.at[p], out_hbm.at[pl.ds(rbase(ci), QC)], sem_out)

    def wait_out(ci, p):
        pltpu.make_async_copy(out_v.at[p], out_hbm.at[pl.ds(rbase(ci), QC)],
                              sem_out).wait()

    def compute(p):
        def q_body(qq, _):
            for h in range(N_HEADS):
                acc0 = jnp.zeros((16,), jnp.float32)
                acc1 = jnp.zeros((16,), jnp.float32)
                for c in range(4):
                    g = c * QC + qq
                    wv = wgt_v[p, c, qq, pl.ds(h * 16, 16)]
                    acca = jnp.zeros((32,), jnp.bfloat16)
                    accb = jnp.zeros((32,), jnp.bfloat16)
                    for lp in range(N_LEVELS * N_POINTS):
                        j = h * 16 + lp
                        wb32 = plsc.bitcast(jnp.full((16,), wv[lp], jnp.int32),
                                            jnp.bfloat16)
                        prod = wb32 * rows_v[p, g, j, pl.ds(0, 32)]
                        if lp % 2 == 0:
                            acca = acca + prod
                        else:
                            accb = accb + prod
                    lo, hi = plsc.unpack(acca + accb,
                                         format=plsc.PackFormat.INTERLEAVED)
                    acc0 = acc0 + lo
                    acc1 = acc1 + hi
                out_v[p, qq, pl.ds(h * 32, 16)] = acc0
                out_v[p, qq, pl.ds(h * 32 + 16, 16)] = acc1
            return 0

        lax.fori_loop(0, QC, q_body, 0)

    start_iw(0, 0)
    wait_iw(0, 0)
    start_gather(0)
    start_iw(1, 1)

    def chunk_body(ci, _):
        p = lax.rem(ci, 2)
        pn = lax.rem(ci + 1, 2)
        wait_gather(p)
        wait_iw(ci + 1, pn)
        start_gather(pn)

        @pl.when(ci >= 2)
        def _():
            wait_out(ci - 2, p)

        compute(p)
        start_out(ci, p)
        start_iw(ci + 2, p)
        return 0

    lax.fori_loop(0, CHUNKS, chunk_body, 0)

    p_end = lax.rem(jnp.int32(CHUNKS), 2)
    wait_gather(p_end)
    wait_iw(CHUNKS, lax.rem(jnp.int32(CHUNKS + 1), 2))
    wait_out(CHUNKS - 2, lax.rem(jnp.int32(CHUNKS - 2), 2))
    wait_out(CHUNKS - 1, lax.rem(jnp.int32(CHUNKS - 1), 2))


def _run_sc(val_rows, idx, wgt):
    mesh = plsc.VectorSubcoreMesh(core_axis_name="c", subcore_axis_name="s",
                                  num_cores=SC_CORES, num_subcores=SC_SUBCORES)
    kern = pl.kernel(
        _sc_body,
        out_type=jax.ShapeDtypeStruct((B_SZ * LEN_Q, D_MODEL), jnp.float32),
        mesh=mesh,
        compiler_params=pltpu.CompilerParams(use_tc_tiling_on_sc=False,
                                             needs_layout_passes=False),
        scratch_types=[
            pltpu.VMEM((2, 4, QC, N_LANE), jnp.int32),
            pltpu.VMEM((2, 4, QC, N_LANE), jnp.int32),
            pltpu.VMEM((2, GROUPS, N_LANE, D_HEAD), jnp.bfloat16),
            pltpu.VMEM((2, QC, D_MODEL), jnp.float32),
            pltpu.SemaphoreType.DMA,
            pltpu.SemaphoreType.DMA,
            pltpu.SemaphoreType.DMA,
        ],
    )
    return kern(val_rows, idx, wgt)


def _outproj_body(x_ref, w_ref, b_ref, o_ref):
    o_ref[...] = jnp.dot(x_ref[...], w_ref[...],
                         preferred_element_type=jnp.float32) + b_ref[...]


def _run_outproj(x, W_out, b_out):
    full = lambda shape: pl.BlockSpec(shape, lambda i: tuple(0 for _ in shape))
    return pl.pallas_call(
        _outproj_body,
        grid=(NB,),
        in_specs=[pl.BlockSpec((QB, D_MODEL), lambda i: (i, 0)),
                  full((D_MODEL, D_MODEL)), full((1, D_MODEL))],
        out_specs=pl.BlockSpec((QB, D_MODEL), lambda i: (i, 0)),
        out_shape=jax.ShapeDtypeStruct((B_SZ * LEN_Q, D_MODEL), jnp.float32),
    )(x, W_out, b_out)


@jax.jit
def kernel(query, reference_points, input_flatten, input_spatial_shapes,
           input_level_start_index, W_off, b_off, W_attn, b_attn,
           W_val, b_val, W_out, b_out):
    W_off3 = W_off.reshape(D_MODEL, N_LANE, 2)
    W_off_x = W_off3[..., 0]
    W_off_y = W_off3[..., 1]
    b_off2 = b_off.reshape(N_LANE, 2)
    b_off_x = b_off2[:, 0].reshape(1, N_LANE)
    b_off_y = b_off2[:, 1].reshape(1, N_LANE)
    rp_l = jnp.broadcast_to(
        reference_points[:, :, None, :, None, :],
        (B_SZ, LEN_Q, N_HEADS, N_LEVELS, N_POINTS, 2),
    ).reshape(B_SZ * LEN_Q, N_LANE, 2)
    rpx_f = rp_l[..., 0]
    rpy_f = rp_l[..., 1]
    query_f = query.reshape(B_SZ * LEN_Q, D_MODEL)
    iff = input_flatten.reshape(B_SZ * LEN_IN, D_MODEL)
    bd = jnp.asarray(BD_NP)
    lanef = jnp.concatenate([jnp.asarray(W_LANE), jnp.asarray(H_LANE)], axis=0)
    lanei = jnp.concatenate([jnp.asarray(WI_LANE), jnp.asarray(HI_LANE),
                             jnp.asarray(BASE_LANE), jnp.asarray(H_LANE_I)], axis=0)

    perm = jnp.asarray(VAL_PERM)
    W_val_p = W_val[:, perm]
    b_val_p = b_val[perm]

    val, idx, wgt = _run_precompute(
        query_f, iff, rpx_f, rpy_f, W_val_p, b_val_p.reshape(1, D_MODEL),
        W_off_x, b_off_x, W_off_y, b_off_y,
        W_attn, b_attn.reshape(1, N_LANE), bd, lanef, lanei)

    val_rows = val.reshape(B_SZ * LEN_IN * N_HEADS, D_HEAD)
    sc_out = _run_sc(val_rows, idx, wgt)

    out = _run_outproj(sc_out, W_out, b_out.reshape(1, D_MODEL))
    return out.reshape(B_SZ, LEN_Q, D_MODEL)
